# Initial kernel scaffold; baseline (speedup 1.0000x reference)
import functools

import jax
import jax.numpy as jnp
from jax.experimental import pallas as pl
from jax.experimental.pallas import tpu as pltpu

NUM_EXPERTS = 8
TOP_K = 2
D_MODEL = 1024
D_FF = 2048
N_TOK = 2048

TOK_BLK = 256
E_PAD = 128


def _router_body(x_ref, wg_ref, comb_ref):
    x = x_ref[...]
    wg = wg_ref[...]
    logits = jnp.dot(x, wg, preferred_element_type=jnp.float32)
    lane = jax.lax.broadcasted_iota(jnp.int32, logits.shape, 1)
    valid = lane < NUM_EXPERTS
    neg = jnp.float32(-1e30)
    logits = jnp.where(valid, logits, neg)
    m = jnp.max(logits, axis=-1, keepdims=True)
    p = jnp.exp(logits - m)
    p = jnp.where(valid, p, 0.0)
    probs = p / jnp.sum(p, axis=-1, keepdims=True)
    w1 = jnp.max(probs, axis=-1, keepdims=True)
    e1 = jnp.min(jnp.where(probs == w1, lane, E_PAD), axis=-1, keepdims=True)
    probs2 = jnp.where(lane == e1, -1.0, probs)
    w2 = jnp.max(probs2, axis=-1, keepdims=True)
    e2 = jnp.min(jnp.where(probs2 == w2, lane, E_PAD), axis=-1, keepdims=True)
    t = w1 + w2
    w1n = w1 / t
    w2n = w2 / t
    comb = jnp.where(lane == e1, w1n, 0.0) + jnp.where(lane == e2, w2n, 0.0)
    comb_ref[...] = comb


def _ffn_body(x_ref, wup_ref, wdn_ref, comb_ref, out_ref, acc_ref):
    e = pl.program_id(0)
    h = jnp.maximum(
        jnp.dot(x_ref[...], wup_ref[0], preferred_element_type=jnp.float32), 0.0
    )
    y = jnp.dot(h, wdn_ref[0], preferred_element_type=jnp.float32)
    lane = jax.lax.broadcasted_iota(jnp.int32, (TOK_BLK, E_PAD), 1)
    cvec = jnp.sum(
        jnp.where(lane == e, comb_ref[...], 0.0), axis=-1, keepdims=True
    )
    contrib = cvec * y

    @pl.when(e == 0)
    def _():
        acc_ref[...] = contrib

    @pl.when(e > 0)
    def _():
        acc_ref[...] = acc_ref[...] + contrib

    @pl.when(e == NUM_EXPERTS - 1)
    def _():
        out_ref[...] = acc_ref[...]


def kernel(x, w_gate, w_up, w_down):
    wg_pad = jnp.pad(w_gate, ((0, 0), (0, E_PAD - NUM_EXPERTS)))

    comb = pl.pallas_call(
        _router_body,
        grid=(N_TOK // TOK_BLK,),
        in_specs=[
            pl.BlockSpec((TOK_BLK, D_MODEL), lambda i: (i, 0)),
            pl.BlockSpec((D_MODEL, E_PAD), lambda i: (0, 0)),
        ],
        out_specs=pl.BlockSpec((TOK_BLK, E_PAD), lambda i: (i, 0)),
        out_shape=jax.ShapeDtypeStruct((N_TOK, E_PAD), jnp.float32),
    )(x, wg_pad)

    out = pl.pallas_call(
        _ffn_body,
        grid=(NUM_EXPERTS, N_TOK // TOK_BLK),
        in_specs=[
            pl.BlockSpec((TOK_BLK, D_MODEL), lambda e, i: (i, 0)),
            pl.BlockSpec((1, D_MODEL, D_FF), lambda e, i: (e, 0, 0)),
            pl.BlockSpec((1, D_FF, D_MODEL), lambda e, i: (e, 0, 0)),
            pl.BlockSpec((TOK_BLK, E_PAD), lambda e, i: (i, 0)),
        ],
        out_specs=pl.BlockSpec((TOK_BLK, D_MODEL), lambda e, i: (i, 0)),
        out_shape=jax.ShapeDtypeStruct((N_TOK, D_MODEL), jnp.float32),
        scratch_shapes=[pltpu.VMEM((TOK_BLK, D_MODEL), jnp.float32)],
    )(x, w_up, w_down, comb)
    return out


# speedup vs baseline: 1.2554x; 1.2554x over previous
import functools

import jax
import jax.numpy as jnp
from jax.experimental import pallas as pl
from jax.experimental.pallas import tpu as pltpu

NUM_EXPERTS = 8
TOP_K = 2
D_MODEL = 1024
D_FF = 2048
N_TOK = 2048

TOK_BLK = 256
E_PAD = 128


def _router_body(x_ref, wg_ref, comb_ref):
    x = x_ref[...]
    wg = wg_ref[...]
    logits = jnp.dot(x, wg, preferred_element_type=jnp.float32)
    lane = jax.lax.broadcasted_iota(jnp.int32, logits.shape, 1)
    valid = lane < NUM_EXPERTS
    neg = jnp.float32(-1e30)
    logits = jnp.where(valid, logits, neg)
    m = jnp.max(logits, axis=-1, keepdims=True)
    p = jnp.exp(logits - m)
    p = jnp.where(valid, p, 0.0)
    probs = p / jnp.sum(p, axis=-1, keepdims=True)
    w1 = jnp.max(probs, axis=-1, keepdims=True)
    e1 = jnp.min(jnp.where(probs == w1, lane, E_PAD), axis=-1, keepdims=True)
    probs2 = jnp.where(lane == e1, -1.0, probs)
    w2 = jnp.max(probs2, axis=-1, keepdims=True)
    e2 = jnp.min(jnp.where(probs2 == w2, lane, E_PAD), axis=-1, keepdims=True)
    t = w1 + w2
    w1n = w1 / t
    w2n = w2 / t
    comb = jnp.where(lane == e1, w1n, 0.0) + jnp.where(lane == e2, w2n, 0.0)
    comb_ref[...] = comb


def _ffn_body(x_ref, wup_ref, wdn_ref, comb_ref, out_ref, acc_ref):
    e = pl.program_id(0)
    i = pl.program_id(1)
    rows = pl.ds(i * TOK_BLK, TOK_BLK)
    h = jnp.maximum(
        jnp.dot(x_ref[...], wup_ref[0], preferred_element_type=jnp.float32), 0.0
    )
    y = jnp.dot(h, wdn_ref[0], preferred_element_type=jnp.float32)
    lane = jax.lax.broadcasted_iota(jnp.int32, (TOK_BLK, E_PAD), 1)
    cvec = jnp.sum(
        jnp.where(lane == e, comb_ref[...], 0.0), axis=-1, keepdims=True
    )
    contrib = cvec * y

    @pl.when(e == 0)
    def _():
        acc_ref[rows, :] = contrib

    @pl.when(e > 0)
    def _():
        acc_ref[rows, :] = acc_ref[rows, :] + contrib

    @pl.when(e == NUM_EXPERTS - 1)
    def _():
        out_ref[...] = acc_ref[rows, :]


def kernel(x, w_gate, w_up, w_down):
    wg_pad = jnp.pad(w_gate, ((0, 0), (0, E_PAD - NUM_EXPERTS)))

    comb = pl.pallas_call(
        _router_body,
        grid=(N_TOK // TOK_BLK,),
        in_specs=[
            pl.BlockSpec((TOK_BLK, D_MODEL), lambda i: (i, 0)),
            pl.BlockSpec((D_MODEL, E_PAD), lambda i: (0, 0)),
        ],
        out_specs=pl.BlockSpec((TOK_BLK, E_PAD), lambda i: (i, 0)),
        out_shape=jax.ShapeDtypeStruct((N_TOK, E_PAD), jnp.float32),
    )(x, wg_pad)

    out = pl.pallas_call(
        _ffn_body,
        grid=(NUM_EXPERTS, N_TOK // TOK_BLK),
        in_specs=[
            pl.BlockSpec((TOK_BLK, D_MODEL), lambda e, i: (i, 0)),
            pl.BlockSpec((1, D_MODEL, D_FF), lambda e, i: (e, 0, 0)),
            pl.BlockSpec((1, D_FF, D_MODEL), lambda e, i: (e, 0, 0)),
            pl.BlockSpec((TOK_BLK, E_PAD), lambda e, i: (i, 0)),
        ],
        out_specs=pl.BlockSpec((TOK_BLK, D_MODEL), lambda e, i: (i, 0)),
        out_shape=jax.ShapeDtypeStruct((N_TOK, D_MODEL), jnp.float32),
        scratch_shapes=[pltpu.VMEM((N_TOK, D_MODEL), jnp.float32)],
    )(x, w_up, w_down, comb)
    return out


# speedup vs baseline: 1.4320x; 1.1407x over previous
import functools

import jax
import jax.numpy as jnp
from jax import lax
from jax.experimental import pallas as pl
from jax.experimental.pallas import tpu as pltpu
from jax.experimental.pallas import tpu_sc as plsc

NUM_EXPERTS = 8
TOP_K = 2
D_MODEL = 1024
D_FF = 2048
N_TOK = 2048
N_PAIR = N_TOK * TOP_K

TOK_BLK = 256
E_PAD = 128

GM_T = 256
GM_NT = N_PAIR // GM_T
GM_S = GM_NT + NUM_EXPERTS - 1

SC_NC = 2
SC_NS = 16
SC_NW = SC_NC * SC_NS
PAIRS_PER_W = N_PAIR // SC_NW
TOKS_PER_W = N_TOK // SC_NW
ROW_CHUNK = 64


def _router_body(x_ref, wg_ref, e1_ref, e2_ref, w1_ref, w2_ref):
    x = x_ref[...]
    logits = jnp.dot(x, wg_ref[...], preferred_element_type=jnp.float32)
    lane = lax.broadcasted_iota(jnp.int32, logits.shape, 1)
    valid = lane < NUM_EXPERTS
    logits = jnp.where(valid, logits, jnp.float32(-1e30))
    m = jnp.max(logits, axis=-1, keepdims=True)
    p = jnp.exp(logits - m)
    p = jnp.where(valid, p, 0.0)
    probs = p / jnp.sum(p, axis=-1, keepdims=True)
    w1 = jnp.max(probs, axis=-1, keepdims=True)
    e1 = jnp.min(jnp.where(probs == w1, lane, E_PAD), axis=-1, keepdims=True)
    probs2 = jnp.where(lane == e1, -1.0, probs)
    w2 = jnp.max(probs2, axis=-1, keepdims=True)
    e2 = jnp.min(jnp.where(probs2 == w2, lane, E_PAD), axis=-1, keepdims=True)
    t = w1 + w2
    e1_ref[...] = e1
    e2_ref[...] = e2
    w1_ref[...] = w1 / t
    w2_ref[...] = w2 / t


def _router(x, w_gate):
    wg_pad = jnp.pad(w_gate, ((0, 0), (0, E_PAD - NUM_EXPERTS)))
    return pl.pallas_call(
        _router_body,
        grid=(N_TOK // TOK_BLK,),
        in_specs=[
            pl.BlockSpec((TOK_BLK, D_MODEL), lambda i: (i, 0)),
            pl.BlockSpec((D_MODEL, E_PAD), lambda i: (0, 0)),
        ],
        out_specs=[
            pl.BlockSpec((TOK_BLK, 1), lambda i: (i, 0)),
            pl.BlockSpec((TOK_BLK, 1), lambda i: (i, 0)),
            pl.BlockSpec((TOK_BLK, 1), lambda i: (i, 0)),
            pl.BlockSpec((TOK_BLK, 1), lambda i: (i, 0)),
        ],
        out_shape=[
            jax.ShapeDtypeStruct((N_TOK, 1), jnp.int32),
            jax.ShapeDtypeStruct((N_TOK, 1), jnp.int32),
            jax.ShapeDtypeStruct((N_TOK, 1), jnp.float32),
            jax.ShapeDtypeStruct((N_TOK, 1), jnp.float32),
        ],
    )(x, wg_pad)


def _dispatch_body(ids_hbm, x_hbm, pos_hbm, offs_hbm, xs_hbm,
                   ids_v, pos_v, idxc0, idxc1, offs_sc_v,
                   hist_a, hist_b, rowbuf, sem):
    wid = lax.axis_index("s") * SC_NC + lax.axis_index("c")
    lane = jnp.arange(16, dtype=jnp.int32)
    zero16 = jnp.zeros((16,), jnp.int32)
    ones16 = jnp.ones((16,), jnp.int32)

    pltpu.sync_copy(ids_hbm, ids_v)

    my_first_vreg = wid * (PAIRS_PER_W // 16)
    for i in range(16):
        hist_a[i] = zero16
        hist_b[i] = zero16

    def hist_step_a(j, carry):
        v = ids_v[pl.ds(j * 16, 16)]
        plsc.addupdate_scatter(hist_a, [lane, v], ones16)
        return carry

    def hist_step_b(j, carry):
        v = ids_v[pl.ds(j * 16, 16)]
        plsc.addupdate_scatter(hist_b, [lane, v], ones16)
        return carry

    lax.fori_loop(0, my_first_vreg, hist_step_a, 0)
    lax.fori_loop(my_first_vreg, N_PAIR // 16, hist_step_b, 0)

    bef = zero16
    tot = zero16
    for i in range(16):
        bef = bef + hist_a[i]
        tot = tot + hist_b[i]
    tot = tot + bef

    incl = plsc.cumsum(tot)
    offs_vec = incl - tot
    base_vec = offs_vec + bef

    for jj in range(PAIRS_PER_W // 16):
        v = ids_v[pl.ds((my_first_vreg + jj) * 16, 16)]
        pos_vec = zero16
        for e in range(NUM_EXPERTS):
            msk = v == e
            mi = msk.astype(jnp.int32)
            rank = plsc.cumsum(mi) - 1
            b_e = jnp.sum(jnp.where(lane == e, base_vec, 0))
            pos_vec = jnp.where(msk, b_e + rank, pos_vec)
            base_vec = base_vec + jnp.where(lane == e, jnp.sum(mi), 0)
        pos_v[pl.ds(jj * 16, 16)] = pos_vec
        if jj < 4:
            idxc0[pl.ds(jj * 16, 16)] = pos_vec
        else:
            idxc1[pl.ds((jj - 4) * 16, 16)] = pos_vec

    pltpu.sync_copy(pos_v, pos_hbm.at[pl.ds(wid * PAIRS_PER_W, PAIRS_PER_W)])

    @pl.when(wid == 0)
    def _():
        offs_sc_v[...] = offs_vec
        pltpu.sync_copy(offs_sc_v, offs_hbm)

    for c, idxc in enumerate((idxc0, idxc1)):
        tok0 = (wid * PAIRS_PER_W + c * ROW_CHUNK) % N_TOK
        pltpu.sync_copy(x_hbm.at[pl.ds(tok0, ROW_CHUNK)], rowbuf)
        pltpu.async_copy(rowbuf, xs_hbm.at[idxc], sem).wait()


def _dispatch(ids, x):
    mesh = plsc.VectorSubcoreMesh(
        core_axis_name="c", subcore_axis_name="s",
        num_cores=SC_NC, num_subcores=SC_NS,
    )
    f = pl.kernel(
        _dispatch_body,
        out_type=[
            jax.ShapeDtypeStruct((N_PAIR,), jnp.int32),
            jax.ShapeDtypeStruct((16,), jnp.int32),
            jax.ShapeDtypeStruct((N_PAIR, D_MODEL), jnp.float32),
        ],
        mesh=mesh,
        scratch_types=[
            pltpu.VMEM((N_PAIR,), jnp.int32),
            pltpu.VMEM((PAIRS_PER_W,), jnp.int32),
            pltpu.VMEM((ROW_CHUNK,), jnp.int32),
            pltpu.VMEM((ROW_CHUNK,), jnp.int32),
            pltpu.VMEM((16,), jnp.int32),
            pltpu.VMEM((16, 16), jnp.int32),
            pltpu.VMEM((16, 16), jnp.int32),
            pltpu.VMEM((ROW_CHUNK, D_MODEL), jnp.float32),
            pltpu.SemaphoreType.DMA,
        ],
        compiler_params=pltpu.CompilerParams(needs_layout_passes=False),
    )
    return f(ids, x)


def _gmm_body(tile_r, grp_r, lo_r, hi_r, xs_ref, wup_ref, wdn_ref, out_ref):
    s = pl.program_id(0)
    t = tile_r[s]
    lo = lo_r[s]
    hi = hi_r[s]
    row = lax.broadcasted_iota(jnp.int32, (GM_T, 1), 0) + t * GM_T
    mask = (row >= lo) & (row < hi)
    h = jnp.maximum(
        jnp.dot(xs_ref[...], wup_ref[0], preferred_element_type=jnp.float32),
        0.0,
    )
    y = jnp.dot(h, wdn_ref[0], preferred_element_type=jnp.float32)
    contrib = jnp.where(mask, y, 0.0)
    t_prev = tile_r[jnp.maximum(s - 1, 0)]
    first = jnp.logical_or(s == 0, t != t_prev)

    @pl.when(first)
    def _():
        out_ref[...] = contrib

    @pl.when(jnp.logical_not(first))
    def _():
        out_ref[...] = out_ref[...] + contrib


def _gmm(xs, w_up, w_down, tile_s, grp_s, lo_s, hi_s):
    grid_spec = pltpu.PrefetchScalarGridSpec(
        num_scalar_prefetch=4,
        grid=(GM_S,),
        in_specs=[
            pl.BlockSpec((GM_T, D_MODEL), lambda s, tr, gr, lr, hr: (tr[s], 0)),
            pl.BlockSpec((1, D_MODEL, D_FF), lambda s, tr, gr, lr, hr: (gr[s], 0, 0)),
            pl.BlockSpec((1, D_FF, D_MODEL), lambda s, tr, gr, lr, hr: (gr[s], 0, 0)),
        ],
        out_specs=pl.BlockSpec((GM_T, D_MODEL), lambda s, tr, gr, lr, hr: (tr[s], 0)),
    )
    return pl.pallas_call(
        _gmm_body,
        grid_spec=grid_spec,
        out_shape=jax.ShapeDtypeStruct((N_PAIR, D_MODEL), jnp.float32),
    )(tile_s, grp_s, lo_s, hi_s, xs, w_up, w_down)


def _schedule(offs):
    offs = offs[:NUM_EXPERTS + 1]
    t_all = jnp.repeat(jnp.arange(GM_NT, dtype=jnp.int32), NUM_EXPERTS)
    e_all = jnp.tile(jnp.arange(NUM_EXPERTS, dtype=jnp.int32), GM_NT)
    lo = jnp.maximum(offs[e_all], t_all * GM_T)
    hi = jnp.minimum(offs[e_all + 1], (t_all + 1) * GM_T)
    valid = hi > lo
    key = jnp.where(valid, t_all * NUM_EXPERTS + e_all, 1 << 20)
    order = jnp.argsort(key)[:GM_S]
    v_s = valid[order]
    e_last = (
        jnp.searchsorted(offs, jnp.int32(N_PAIR - 1), side="right") - 1
    ).astype(jnp.int32)
    tile_s = jnp.where(v_s, t_all[order], GM_NT - 1)
    grp_s = jnp.where(v_s, e_all[order], e_last)
    lo_s = jnp.where(v_s, lo[order], 0)
    hi_s = jnp.where(v_s, hi[order], 0)
    return tile_s, grp_s, lo_s, hi_s


def _combine_sc_body(y_hbm, pos_hbm, r1_hbm, r2_hbm, idxa, idxb, rowbuf, sem):
    wid = lax.axis_index("s") * SC_NC + lax.axis_index("c")
    tb = wid * TOKS_PER_W
    pltpu.sync_copy(pos_hbm.at[pl.ds(tb, TOKS_PER_W)], idxa)
    pltpu.sync_copy(pos_hbm.at[pl.ds(N_TOK + tb, TOKS_PER_W)], idxb)
    pltpu.async_copy(y_hbm.at[idxa], rowbuf, sem).wait()
    pltpu.sync_copy(rowbuf, r1_hbm.at[pl.ds(tb, TOKS_PER_W)])
    pltpu.async_copy(y_hbm.at[idxb], rowbuf, sem).wait()
    pltpu.sync_copy(rowbuf, r2_hbm.at[pl.ds(tb, TOKS_PER_W)])


def _combine_sc(y_sorted, pos):
    mesh = plsc.VectorSubcoreMesh(
        core_axis_name="c", subcore_axis_name="s",
        num_cores=SC_NC, num_subcores=SC_NS,
    )
    f = pl.kernel(
        _combine_sc_body,
        out_type=[
            jax.ShapeDtypeStruct((N_TOK, D_MODEL), jnp.float32),
            jax.ShapeDtypeStruct((N_TOK, D_MODEL), jnp.float32),
        ],
        mesh=mesh,
        scratch_types=[
            pltpu.VMEM((TOKS_PER_W,), jnp.int32),
            pltpu.VMEM((TOKS_PER_W,), jnp.int32),
            pltpu.VMEM((TOKS_PER_W, D_MODEL), jnp.float32),
            pltpu.SemaphoreType.DMA,
        ],
        compiler_params=pltpu.CompilerParams(needs_layout_passes=False),
    )
    return f(y_sorted, pos)


def _combine_tc_body(r1_ref, r2_ref, w1_ref, w2_ref, out_ref):
    out_ref[...] = w1_ref[...] * r1_ref[...] + w2_ref[...] * r2_ref[...]


def _combine_tc(r1, r2, w1n, w2n):
    return pl.pallas_call(
        _combine_tc_body,
        grid=(N_TOK // TOK_BLK,),
        in_specs=[
            pl.BlockSpec((TOK_BLK, D_MODEL), lambda i: (i, 0)),
            pl.BlockSpec((TOK_BLK, D_MODEL), lambda i: (i, 0)),
            pl.BlockSpec((TOK_BLK, 1), lambda i: (i, 0)),
            pl.BlockSpec((TOK_BLK, 1), lambda i: (i, 0)),
        ],
        out_specs=pl.BlockSpec((TOK_BLK, D_MODEL), lambda i: (i, 0)),
        out_shape=jax.ShapeDtypeStruct((N_TOK, D_MODEL), jnp.float32),
    )(r1, r2, w1n, w2n)


def kernel(x, w_gate, w_up, w_down):
    e1, e2, w1n, w2n = _router(x, w_gate)
    ids = jnp.concatenate([e1[:, 0], e2[:, 0]])
    pos, offs, xs = _dispatch(ids, x)
    tile_s, grp_s, lo_s, hi_s = _schedule(offs)
    y_sorted = _gmm(xs, w_up, w_down, tile_s, grp_s, lo_s, hi_s)
    r1, r2 = _combine_sc(y_sorted, pos)
    return _combine_tc(r1, r2, w1n, w2n)


# speedup vs baseline: 1.5811x; 1.1041x over previous
import functools

import jax
import jax.numpy as jnp
from jax import lax
from jax.experimental import pallas as pl
from jax.experimental.pallas import tpu as pltpu
from jax.experimental.pallas import tpu_sc as plsc

NUM_EXPERTS = 8
TOP_K = 2
D_MODEL = 1024
D_FF = 2048
N_TOK = 2048
N_PAIR = N_TOK * TOP_K

TOK_BLK = 256
E_PAD = 128

GM_T = 256
GM_NT = N_PAIR // GM_T
GM_S = GM_NT + NUM_EXPERTS - 1
SCHED_N = 32

SC_NC = 2
SC_NS = 16
SC_NW = SC_NC * SC_NS
PAIRS_PER_W = N_PAIR // SC_NW
TOKS_PER_W = N_TOK // SC_NW
ROW_CHUNK = 32
N_CHUNK = PAIRS_PER_W // ROW_CHUNK


def _router_body(x_ref, wg_ref, e1_ref, e2_ref, w1_ref, w2_ref):
    x = x_ref[...]
    logits = jnp.dot(x, wg_ref[...], preferred_element_type=jnp.float32)
    lane = lax.broadcasted_iota(jnp.int32, logits.shape, 1)
    valid = lane < NUM_EXPERTS
    logits = jnp.where(valid, logits, jnp.float32(-1e30))
    m = jnp.max(logits, axis=-1, keepdims=True)
    p = jnp.exp(logits - m)
    p = jnp.where(valid, p, 0.0)
    probs = p / jnp.sum(p, axis=-1, keepdims=True)
    w1 = jnp.max(probs, axis=-1, keepdims=True)
    e1 = jnp.min(jnp.where(probs == w1, lane, E_PAD), axis=-1, keepdims=True)
    probs2 = jnp.where(lane == e1, -1.0, probs)
    w2 = jnp.max(probs2, axis=-1, keepdims=True)
    e2 = jnp.min(jnp.where(probs2 == w2, lane, E_PAD), axis=-1, keepdims=True)
    t = w1 + w2
    e1_ref[...] = e1
    e2_ref[...] = e2
    w1_ref[...] = w1 / t
    w2_ref[...] = w2 / t


def _router(x, w_gate):
    wg_pad = jnp.pad(w_gate, ((0, 0), (0, E_PAD - NUM_EXPERTS)))
    return pl.pallas_call(
        _router_body,
        grid=(N_TOK // TOK_BLK,),
        in_specs=[
            pl.BlockSpec((TOK_BLK, D_MODEL), lambda i: (i, 0)),
            pl.BlockSpec((D_MODEL, E_PAD), lambda i: (0, 0)),
        ],
        out_specs=[
            pl.BlockSpec((TOK_BLK, 1), lambda i: (i, 0)),
            pl.BlockSpec((TOK_BLK, 1), lambda i: (i, 0)),
            pl.BlockSpec((TOK_BLK, 1), lambda i: (i, 0)),
            pl.BlockSpec((TOK_BLK, 1), lambda i: (i, 0)),
        ],
        out_shape=[
            jax.ShapeDtypeStruct((N_TOK, 1), jnp.int32),
            jax.ShapeDtypeStruct((N_TOK, 1), jnp.int32),
            jax.ShapeDtypeStruct((N_TOK, 1), jnp.float32),
            jax.ShapeDtypeStruct((N_TOK, 1), jnp.float32),
        ],
    )(x, wg_pad)


def _dispatch_body(e1_hbm, e2_hbm, x_hbm,
                   pos_hbm, st_hbm, sg_hbm, slo_hbm, shi_hbm, xs_hbm,
                   ids_v, pos_v, idx0, idx1, idx2, idx3,
                   st_v, sg_v, slo_v, shi_v,
                   hist_a, hist_b, buf_a, buf_b, sem_r, sem_w):
    wid = lax.axis_index("s") * SC_NC + lax.axis_index("c")
    lane = jnp.arange(16, dtype=jnp.int32)
    zero16 = jnp.zeros((16,), jnp.int32)
    ones16 = jnp.ones((16,), jnp.int32)
    idx_refs = (idx0, idx1, idx2, idx3)
    bufs = (buf_a, buf_b)

    def row0(c):
        return (wid * PAIRS_PER_W + c * ROW_CHUNK) % N_TOK

    rd = [pltpu.async_copy(x_hbm.at[pl.ds(row0(c), ROW_CHUNK)], bufs[c % 2],
                           sem_r) for c in range(2)]

    pltpu.sync_copy(e1_hbm, ids_v.at[pl.ds(0, N_TOK)])
    pltpu.sync_copy(e2_hbm, ids_v.at[pl.ds(N_TOK, N_TOK)])

    my_first_vreg = wid * (PAIRS_PER_W // 16)
    for i in range(16):
        hist_a[i] = zero16
        hist_b[i] = zero16

    def hist_step_a(j, carry):
        v = ids_v[pl.ds(j * 16, 16)]
        plsc.addupdate_scatter(hist_a, [lane, v], ones16)
        return carry

    def hist_step_b(j, carry):
        v = ids_v[pl.ds(j * 16, 16)]
        plsc.addupdate_scatter(hist_b, [lane, v], ones16)
        return carry

    lax.fori_loop(0, my_first_vreg, hist_step_a, 0)
    lax.fori_loop(my_first_vreg, N_PAIR // 16, hist_step_b, 0)

    bef = zero16
    tot = zero16
    for i in range(16):
        bef = bef + hist_a[i]
        tot = tot + hist_b[i]
    tot = tot + bef

    incl = plsc.cumsum(tot)
    offs_vec = incl - tot
    base_vec = offs_vec + bef

    for jj in range(PAIRS_PER_W // 16):
        v = ids_v[pl.ds((my_first_vreg + jj) * 16, 16)]
        pos_vec = zero16
        for e in range(NUM_EXPERTS):
            msk = v == e
            mi = msk.astype(jnp.int32)
            rank = plsc.cumsum(mi) - 1
            b_e = jnp.sum(jnp.where(lane == e, base_vec, 0))
            pos_vec = jnp.where(msk, b_e + rank, pos_vec)
            base_vec = base_vec + jnp.where(lane == e, jnp.sum(mi), 0)
        pos_v[pl.ds(jj * 16, 16)] = pos_vec
        idx_refs[jj // 2][pl.ds((jj % 2) * 16, 16)] = pos_vec

    pltpu.sync_copy(pos_v, pos_hbm.at[pl.ds(wid * PAIRS_PER_W, PAIRS_PER_W)])

    @pl.when(wid == 0)
    def _():
        e_last = jnp.sum((incl <= N_PAIR - 1).astype(jnp.int32))
        for r in range(SCHED_N // 16):
            st_v[pl.ds(r * 16, 16)] = zero16 + (GM_NT - 1)
            sg_v[pl.ds(r * 16, 16)] = zero16 + e_last
            slo_v[pl.ds(r * 16, 16)] = zero16
            shi_v[pl.ds(r * 16, 16)] = zero16
        running = jnp.int32(0)
        for e in range(NUM_EXPERTS):
            o_lo = jnp.sum(jnp.where(lane == e, offs_vec, 0))
            o_hi = jnp.sum(jnp.where(lane == e, incl, 0))
            lo_v = jnp.maximum(o_lo, lane * GM_T)
            hi_v = jnp.minimum(o_hi, (lane + 1) * GM_T)
            valid = hi_v > lo_v
            vi = valid.astype(jnp.int32)
            posn = running + plsc.cumsum(vi) - 1
            plsc.store_scatter(st_v, [posn], lane, mask=valid)
            plsc.store_scatter(sg_v, [posn], zero16 + e, mask=valid)
            plsc.store_scatter(slo_v, [posn], lo_v, mask=valid)
            plsc.store_scatter(shi_v, [posn], hi_v, mask=valid)
            running = running + jnp.sum(vi)
        pltpu.sync_copy(st_v, st_hbm)
        pltpu.sync_copy(sg_v, sg_hbm)
        pltpu.sync_copy(slo_v, slo_hbm)
        pltpu.sync_copy(shi_v, shi_hbm)

    wr = [None] * N_CHUNK
    for c in range(N_CHUNK):
        if c >= 2:
            wr[c - 2].wait()
            rd.append(pltpu.async_copy(
                x_hbm.at[pl.ds(row0(c), ROW_CHUNK)], bufs[c % 2], sem_r))
        rd[c].wait()
        wr[c] = pltpu.async_copy(bufs[c % 2], xs_hbm.at[idx_refs[c]], sem_w)
    wr[N_CHUNK - 2].wait()
    wr[N_CHUNK - 1].wait()


def _dispatch(e1, e2, x):
    mesh = plsc.VectorSubcoreMesh(
        core_axis_name="c", subcore_axis_name="s",
        num_cores=SC_NC, num_subcores=SC_NS,
    )
    f = pl.kernel(
        _dispatch_body,
        out_type=[
            jax.ShapeDtypeStruct((N_PAIR,), jnp.int32),
            jax.ShapeDtypeStruct((SCHED_N,), jnp.int32),
            jax.ShapeDtypeStruct((SCHED_N,), jnp.int32),
            jax.ShapeDtypeStruct((SCHED_N,), jnp.int32),
            jax.ShapeDtypeStruct((SCHED_N,), jnp.int32),
            jax.ShapeDtypeStruct((N_PAIR, D_MODEL), jnp.float32),
        ],
        mesh=mesh,
        scratch_types=[
            pltpu.VMEM((N_PAIR,), jnp.int32),
            pltpu.VMEM((PAIRS_PER_W,), jnp.int32),
            pltpu.VMEM((ROW_CHUNK,), jnp.int32),
            pltpu.VMEM((ROW_CHUNK,), jnp.int32),
            pltpu.VMEM((ROW_CHUNK,), jnp.int32),
            pltpu.VMEM((ROW_CHUNK,), jnp.int32),
            pltpu.VMEM((SCHED_N,), jnp.int32),
            pltpu.VMEM((SCHED_N,), jnp.int32),
            pltpu.VMEM((SCHED_N,), jnp.int32),
            pltpu.VMEM((SCHED_N,), jnp.int32),
            pltpu.VMEM((16, 16), jnp.int32),
            pltpu.VMEM((16, 16), jnp.int32),
            pltpu.VMEM((ROW_CHUNK, D_MODEL), jnp.float32),
            pltpu.VMEM((ROW_CHUNK, D_MODEL), jnp.float32),
            pltpu.SemaphoreType.DMA,
            pltpu.SemaphoreType.DMA,
        ],
        compiler_params=pltpu.CompilerParams(needs_layout_passes=False),
    )
    return f(e1, e2, x)


def _gmm_body(tile_r, grp_r, lo_r, hi_r, xs_ref, wup_ref, wdn_ref, out_ref):
    s = pl.program_id(0)
    t = tile_r[s]
    lo = lo_r[s]
    hi = hi_r[s]
    row = lax.broadcasted_iota(jnp.int32, (GM_T, 1), 0) + t * GM_T
    mask = (row >= lo) & (row < hi)
    xb = xs_ref[...].astype(jnp.bfloat16)
    wup = wup_ref[0].astype(jnp.bfloat16)
    wdn = wdn_ref[0].astype(jnp.bfloat16)
    h = jnp.maximum(jnp.dot(xb, wup, preferred_element_type=jnp.float32), 0.0)
    y = jnp.dot(h.astype(jnp.bfloat16), wdn,
                preferred_element_type=jnp.float32)
    contrib = jnp.where(mask, y, 0.0)
    t_prev = tile_r[jnp.maximum(s - 1, 0)]
    first = jnp.logical_or(s == 0, t != t_prev)

    @pl.when(first)
    def _():
        out_ref[...] = contrib

    @pl.when(jnp.logical_not(first))
    def _():
        out_ref[...] = out_ref[...] + contrib


def _gmm(xs, w_up, w_down, tile_s, grp_s, lo_s, hi_s):
    grid_spec = pltpu.PrefetchScalarGridSpec(
        num_scalar_prefetch=4,
        grid=(GM_S,),
        in_specs=[
            pl.BlockSpec((GM_T, D_MODEL), lambda s, tr, gr, lr, hr: (tr[s], 0)),
            pl.BlockSpec((1, D_MODEL, D_FF), lambda s, tr, gr, lr, hr: (gr[s], 0, 0)),
            pl.BlockSpec((1, D_FF, D_MODEL), lambda s, tr, gr, lr, hr: (gr[s], 0, 0)),
        ],
        out_specs=pl.BlockSpec((GM_T, D_MODEL), lambda s, tr, gr, lr, hr: (tr[s], 0)),
    )
    return pl.pallas_call(
        _gmm_body,
        grid_spec=grid_spec,
        out_shape=jax.ShapeDtypeStruct((N_PAIR, D_MODEL), jnp.float32),
    )(tile_s, grp_s, lo_s, hi_s, xs, w_up, w_down)


def _combine_sc_body(y_hbm, pos_hbm, r1_hbm, r2_hbm,
                     idxa, idxb, buf_a, buf_b, sem_g, sem_w):
    wid = lax.axis_index("s") * SC_NC + lax.axis_index("c")
    tb = wid * TOKS_PER_W
    pltpu.sync_copy(pos_hbm.at[pl.ds(tb, TOKS_PER_W)], idxa)
    pltpu.sync_copy(pos_hbm.at[pl.ds(N_TOK + tb, TOKS_PER_W)], idxb)
    bufs = (buf_a, buf_b)
    plan = [(idxa, r1_hbm, 0), (idxa, r1_hbm, 1),
            (idxb, r2_hbm, 0), (idxb, r2_hbm, 1)]
    gd = [None] * 4
    wd = [None] * 4
    for i, (idx, dst, c) in enumerate(plan):
        if i >= 2:
            wd[i - 2].wait()
        gd[i] = pltpu.async_copy(
            y_hbm.at[idx.at[pl.ds(c * ROW_CHUNK, ROW_CHUNK)]],
            bufs[i % 2], sem_g)
        gd[i].wait()
        wd[i] = pltpu.async_copy(
            bufs[i % 2], dst.at[pl.ds(tb + c * ROW_CHUNK, ROW_CHUNK)], sem_w)
    wd[2].wait()
    wd[3].wait()


def _combine_sc(y_sorted, pos):
    mesh = plsc.VectorSubcoreMesh(
        core_axis_name="c", subcore_axis_name="s",
        num_cores=SC_NC, num_subcores=SC_NS,
    )
    f = pl.kernel(
        _combine_sc_body,
        out_type=[
            jax.ShapeDtypeStruct((N_TOK, D_MODEL), jnp.float32),
            jax.ShapeDtypeStruct((N_TOK, D_MODEL), jnp.float32),
        ],
        mesh=mesh,
        scratch_types=[
            pltpu.VMEM((TOKS_PER_W,), jnp.int32),
            pltpu.VMEM((TOKS_PER_W,), jnp.int32),
            pltpu.VMEM((ROW_CHUNK, D_MODEL), jnp.float32),
            pltpu.VMEM((ROW_CHUNK, D_MODEL), jnp.float32),
            pltpu.SemaphoreType.DMA,
            pltpu.SemaphoreType.DMA,
        ],
        compiler_params=pltpu.CompilerParams(needs_layout_passes=False),
    )
    return f(y_sorted, pos)


def _combine_tc_body(r1_ref, r2_ref, w1_ref, w2_ref, out_ref):
    out_ref[...] = w1_ref[...] * r1_ref[...] + w2_ref[...] * r2_ref[...]


def _combine_tc(r1, r2, w1n, w2n):
    return pl.pallas_call(
        _combine_tc_body,
        grid=(N_TOK // TOK_BLK,),
        in_specs=[
            pl.BlockSpec((TOK_BLK, D_MODEL), lambda i: (i, 0)),
            pl.BlockSpec((TOK_BLK, D_MODEL), lambda i: (i, 0)),
            pl.BlockSpec((TOK_BLK, 1), lambda i: (i, 0)),
            pl.BlockSpec((TOK_BLK, 1), lambda i: (i, 0)),
        ],
        out_specs=pl.BlockSpec((TOK_BLK, D_MODEL), lambda i: (i, 0)),
        out_shape=jax.ShapeDtypeStruct((N_TOK, D_MODEL), jnp.float32),
    )(r1, r2, w1n, w2n)


def kernel(x, w_gate, w_up, w_down):
    e1, e2, w1n, w2n = _router(x, w_gate)
    pos, st, sg, slo, shi, xs = _dispatch(
        e1.reshape(N_TOK), e2.reshape(N_TOK), x)
    y_sorted = _gmm(xs, w_up, w_down, st, sg, slo, shi)
    r1, r2 = _combine_sc(y_sorted, pos)
    return _combine_tc(r1, r2, w1n, w2n)


# speedup vs baseline: 1.5832x; 1.0013x over previous
import functools

import jax
import jax.numpy as jnp
from jax import lax
from jax.experimental import pallas as pl
from jax.experimental.pallas import tpu as pltpu
from jax.experimental.pallas import tpu_sc as plsc

NUM_EXPERTS = 8
TOP_K = 2
D_MODEL = 1024
D_FF = 2048
N_TOK = 2048
N_PAIR = N_TOK * TOP_K

TOK_BLK = 256
E_PAD = 128

GM_T = 256
GM_NT = N_PAIR // GM_T
GM_S = GM_NT + NUM_EXPERTS - 1
SCHED_N = 32

SC_NC = 2
SC_NS = 16
SC_NW = SC_NC * SC_NS
PAIRS_PER_W = N_PAIR // SC_NW
TOKS_PER_W = N_TOK // SC_NW
ROW_CHUNK = 32
N_CHUNK = PAIRS_PER_W // ROW_CHUNK


def _router_body(x_ref, wg_ref, e1_ref, e2_ref, w1_ref, w2_ref):
    x = x_ref[...]
    logits = jnp.dot(x, wg_ref[...], preferred_element_type=jnp.float32)
    lane = lax.broadcasted_iota(jnp.int32, logits.shape, 1)
    valid = lane < NUM_EXPERTS
    logits = jnp.where(valid, logits, jnp.float32(-1e30))
    m = jnp.max(logits, axis=-1, keepdims=True)
    p = jnp.exp(logits - m)
    p = jnp.where(valid, p, 0.0)
    probs = p / jnp.sum(p, axis=-1, keepdims=True)
    w1 = jnp.max(probs, axis=-1, keepdims=True)
    e1 = jnp.min(jnp.where(probs == w1, lane, E_PAD), axis=-1, keepdims=True)
    probs2 = jnp.where(lane == e1, -1.0, probs)
    w2 = jnp.max(probs2, axis=-1, keepdims=True)
    e2 = jnp.min(jnp.where(probs2 == w2, lane, E_PAD), axis=-1, keepdims=True)
    t = w1 + w2
    e1_ref[...] = e1
    e2_ref[...] = e2
    w1_ref[...] = w1 / t
    w2_ref[...] = w2 / t


def _router(x, w_gate):
    wg_pad = jnp.pad(w_gate, ((0, 0), (0, E_PAD - NUM_EXPERTS)))
    return pl.pallas_call(
        _router_body,
        grid=(N_TOK // TOK_BLK,),
        in_specs=[
            pl.BlockSpec((TOK_BLK, D_MODEL), lambda i: (i, 0)),
            pl.BlockSpec((D_MODEL, E_PAD), lambda i: (0, 0)),
        ],
        out_specs=[
            pl.BlockSpec((TOK_BLK, 1), lambda i: (i, 0)),
            pl.BlockSpec((TOK_BLK, 1), lambda i: (i, 0)),
            pl.BlockSpec((TOK_BLK, 1), lambda i: (i, 0)),
            pl.BlockSpec((TOK_BLK, 1), lambda i: (i, 0)),
        ],
        out_shape=[
            jax.ShapeDtypeStruct((N_TOK, 1), jnp.int32),
            jax.ShapeDtypeStruct((N_TOK, 1), jnp.int32),
            jax.ShapeDtypeStruct((N_TOK, 1), jnp.float32),
            jax.ShapeDtypeStruct((N_TOK, 1), jnp.float32),
        ],
    )(x, wg_pad)


def _dispatch_body(e1_hbm, e2_hbm, x_hbm,
                   pos_hbm, st_hbm, sg_hbm, slo_hbm, shi_hbm, xs_hbm,
                   ids_v, pos_v, idx0, idx1, idx2, idx3,
                   st_v, sg_v, slo_v, shi_v,
                   hist_a, hist_b, buf_a, buf_b, sem_r, sem_w):
    wid = lax.axis_index("s") * SC_NC + lax.axis_index("c")
    lane = jnp.arange(16, dtype=jnp.int32)
    zero16 = jnp.zeros((16,), jnp.int32)
    ones16 = jnp.ones((16,), jnp.int32)
    idx_refs = (idx0, idx1, idx2, idx3)
    bufs = (buf_a, buf_b)

    def row0(c):
        return (wid * PAIRS_PER_W + c * ROW_CHUNK) % N_TOK

    rd = [pltpu.async_copy(x_hbm.at[pl.ds(row0(c), ROW_CHUNK)], bufs[c % 2],
                           sem_r) for c in range(2)]

    pltpu.sync_copy(e1_hbm, ids_v.at[pl.ds(0, N_TOK)])
    pltpu.sync_copy(e2_hbm, ids_v.at[pl.ds(N_TOK, N_TOK)])

    my_first_vreg = wid * (PAIRS_PER_W // 16)
    for i in range(16):
        hist_a[i] = zero16
        hist_b[i] = zero16

    def hist_step_a(j, carry):
        v = ids_v[pl.ds(j * 16, 16)]
        plsc.addupdate_scatter(hist_a, [lane, v], ones16)
        return carry

    def hist_step_b(j, carry):
        v = ids_v[pl.ds(j * 16, 16)]
        plsc.addupdate_scatter(hist_b, [lane, v], ones16)
        return carry

    lax.fori_loop(0, my_first_vreg, hist_step_a, 0)
    lax.fori_loop(my_first_vreg, N_PAIR // 16, hist_step_b, 0)

    bef = zero16
    tot = zero16
    for i in range(16):
        bef = bef + hist_a[i]
        tot = tot + hist_b[i]
    tot = tot + bef

    incl = plsc.cumsum(tot)
    offs_vec = incl - tot
    base_vec = offs_vec + bef

    for jj in range(PAIRS_PER_W // 16):
        v = ids_v[pl.ds((my_first_vreg + jj) * 16, 16)]
        pos_vec = zero16
        for e in range(NUM_EXPERTS):
            msk = v == e
            mi = msk.astype(jnp.int32)
            rank = plsc.cumsum(mi) - 1
            b_e = jnp.sum(jnp.where(lane == e, base_vec, 0))
            pos_vec = jnp.where(msk, b_e + rank, pos_vec)
            base_vec = base_vec + jnp.where(lane == e, jnp.sum(mi), 0)
        pos_v[pl.ds(jj * 16, 16)] = pos_vec
        idx_refs[jj // 2][pl.ds((jj % 2) * 16, 16)] = pos_vec

    pltpu.sync_copy(pos_v, pos_hbm.at[pl.ds(wid * PAIRS_PER_W, PAIRS_PER_W)])

    @pl.when(wid == 0)
    def _():
        e_last = jnp.sum((incl <= N_PAIR - 1).astype(jnp.int32))
        for r in range(SCHED_N // 16):
            st_v[pl.ds(r * 16, 16)] = zero16 + (GM_NT - 1)
            sg_v[pl.ds(r * 16, 16)] = zero16 + e_last
            slo_v[pl.ds(r * 16, 16)] = zero16
            shi_v[pl.ds(r * 16, 16)] = zero16
        running = jnp.int32(0)
        for e in range(NUM_EXPERTS):
            o_lo = jnp.sum(jnp.where(lane == e, offs_vec, 0))
            o_hi = jnp.sum(jnp.where(lane == e, incl, 0))
            lo_v = jnp.maximum(o_lo, lane * GM_T)
            hi_v = jnp.minimum(o_hi, (lane + 1) * GM_T)
            valid = hi_v > lo_v
            vi = valid.astype(jnp.int32)
            posn = running + plsc.cumsum(vi) - 1
            plsc.store_scatter(st_v, [posn], lane, mask=valid)
            plsc.store_scatter(sg_v, [posn], zero16 + e, mask=valid)
            plsc.store_scatter(slo_v, [posn], lo_v, mask=valid)
            plsc.store_scatter(shi_v, [posn], hi_v, mask=valid)
            running = running + jnp.sum(vi)
        pltpu.sync_copy(st_v, st_hbm)
        pltpu.sync_copy(sg_v, sg_hbm)
        pltpu.sync_copy(slo_v, slo_hbm)
        pltpu.sync_copy(shi_v, shi_hbm)

    wr = [None] * N_CHUNK
    for c in range(N_CHUNK):
        if c >= 2:
            wr[c - 2].wait()
            rd.append(pltpu.async_copy(
                x_hbm.at[pl.ds(row0(c), ROW_CHUNK)], bufs[c % 2], sem_r))
        rd[c].wait()
        wr[c] = pltpu.async_copy(bufs[c % 2], xs_hbm.at[idx_refs[c]], sem_w)
    wr[N_CHUNK - 2].wait()
    wr[N_CHUNK - 1].wait()


def _dispatch(e1, e2, x):
    mesh = plsc.VectorSubcoreMesh(
        core_axis_name="c", subcore_axis_name="s",
        num_cores=SC_NC, num_subcores=SC_NS,
    )
    f = pl.kernel(
        _dispatch_body,
        out_type=[
            jax.ShapeDtypeStruct((N_PAIR,), jnp.int32),
            jax.ShapeDtypeStruct((SCHED_N,), jnp.int32),
            jax.ShapeDtypeStruct((SCHED_N,), jnp.int32),
            jax.ShapeDtypeStruct((SCHED_N,), jnp.int32),
            jax.ShapeDtypeStruct((SCHED_N,), jnp.int32),
            jax.ShapeDtypeStruct((N_PAIR, D_MODEL), jnp.float32),
        ],
        mesh=mesh,
        scratch_types=[
            pltpu.VMEM((N_PAIR,), jnp.int32),
            pltpu.VMEM((PAIRS_PER_W,), jnp.int32),
            pltpu.VMEM((ROW_CHUNK,), jnp.int32),
            pltpu.VMEM((ROW_CHUNK,), jnp.int32),
            pltpu.VMEM((ROW_CHUNK,), jnp.int32),
            pltpu.VMEM((ROW_CHUNK,), jnp.int32),
            pltpu.VMEM((SCHED_N,), jnp.int32),
            pltpu.VMEM((SCHED_N,), jnp.int32),
            pltpu.VMEM((SCHED_N,), jnp.int32),
            pltpu.VMEM((SCHED_N,), jnp.int32),
            pltpu.VMEM((16, 16), jnp.int32),
            pltpu.VMEM((16, 16), jnp.int32),
            pltpu.VMEM((ROW_CHUNK, D_MODEL), jnp.float32),
            pltpu.VMEM((ROW_CHUNK, D_MODEL), jnp.float32),
            pltpu.SemaphoreType.DMA,
            pltpu.SemaphoreType.DMA,
        ],
        compiler_params=pltpu.CompilerParams(needs_layout_passes=False),
    )
    return f(e1, e2, x)


def _gmm_body(tile_r, grp_r, lo_r, hi_r, xs_ref, wup_ref, wdn_ref, out_ref):
    s = pl.program_id(0)
    t = tile_r[s]
    lo = lo_r[s]
    hi = hi_r[s]
    row = lax.broadcasted_iota(jnp.int32, (GM_T, 1), 0) + t * GM_T
    mask = (row >= lo) & (row < hi)
    h = jnp.maximum(
        jnp.dot(xs_ref[...], wup_ref[0], preferred_element_type=jnp.float32),
        0.0,
    )
    y = jnp.dot(h, wdn_ref[0], preferred_element_type=jnp.float32)
    contrib = jnp.where(mask, y, 0.0)
    t_prev = tile_r[jnp.maximum(s - 1, 0)]
    first = jnp.logical_or(s == 0, t != t_prev)

    @pl.when(first)
    def _():
        out_ref[...] = contrib

    @pl.when(jnp.logical_not(first))
    def _():
        out_ref[...] = out_ref[...] + contrib


def _gmm(xs, w_up, w_down, tile_s, grp_s, lo_s, hi_s):
    grid_spec = pltpu.PrefetchScalarGridSpec(
        num_scalar_prefetch=4,
        grid=(GM_S,),
        in_specs=[
            pl.BlockSpec((GM_T, D_MODEL), lambda s, tr, gr, lr, hr: (tr[s], 0)),
            pl.BlockSpec((1, D_MODEL, D_FF), lambda s, tr, gr, lr, hr: (gr[s], 0, 0)),
            pl.BlockSpec((1, D_FF, D_MODEL), lambda s, tr, gr, lr, hr: (gr[s], 0, 0)),
        ],
        out_specs=pl.BlockSpec((GM_T, D_MODEL), lambda s, tr, gr, lr, hr: (tr[s], 0)),
    )
    return pl.pallas_call(
        _gmm_body,
        grid_spec=grid_spec,
        out_shape=jax.ShapeDtypeStruct((N_PAIR, D_MODEL), jnp.float32),
    )(tile_s, grp_s, lo_s, hi_s, xs, w_up, w_down)


def _combine_sc_body(y_hbm, pos_hbm, r1_hbm, r2_hbm,
                     idxa, idxb, buf_a, buf_b, sem_g, sem_w):
    wid = lax.axis_index("s") * SC_NC + lax.axis_index("c")
    tb = wid * TOKS_PER_W
    pltpu.sync_copy(pos_hbm.at[pl.ds(tb, TOKS_PER_W)], idxa)
    pltpu.sync_copy(pos_hbm.at[pl.ds(N_TOK + tb, TOKS_PER_W)], idxb)
    bufs = (buf_a, buf_b)
    plan = [(idxa, r1_hbm, 0), (idxa, r1_hbm, 1),
            (idxb, r2_hbm, 0), (idxb, r2_hbm, 1)]
    gd = [None] * 4
    wd = [None] * 4
    for i, (idx, dst, c) in enumerate(plan):
        if i >= 2:
            wd[i - 2].wait()
        gd[i] = pltpu.async_copy(
            y_hbm.at[idx.at[pl.ds(c * ROW_CHUNK, ROW_CHUNK)]],
            bufs[i % 2], sem_g)
        gd[i].wait()
        wd[i] = pltpu.async_copy(
            bufs[i % 2], dst.at[pl.ds(tb + c * ROW_CHUNK, ROW_CHUNK)], sem_w)
    wd[2].wait()
    wd[3].wait()


def _combine_sc(y_sorted, pos):
    mesh = plsc.VectorSubcoreMesh(
        core_axis_name="c", subcore_axis_name="s",
        num_cores=SC_NC, num_subcores=SC_NS,
    )
    f = pl.kernel(
        _combine_sc_body,
        out_type=[
            jax.ShapeDtypeStruct((N_TOK, D_MODEL), jnp.float32),
            jax.ShapeDtypeStruct((N_TOK, D_MODEL), jnp.float32),
        ],
        mesh=mesh,
        scratch_types=[
            pltpu.VMEM((TOKS_PER_W,), jnp.int32),
            pltpu.VMEM((TOKS_PER_W,), jnp.int32),
            pltpu.VMEM((ROW_CHUNK, D_MODEL), jnp.float32),
            pltpu.VMEM((ROW_CHUNK, D_MODEL), jnp.float32),
            pltpu.SemaphoreType.DMA,
            pltpu.SemaphoreType.DMA,
        ],
        compiler_params=pltpu.CompilerParams(needs_layout_passes=False),
    )
    return f(y_sorted, pos)


def _combine_tc_body(r1_ref, r2_ref, w1_ref, w2_ref, out_ref):
    out_ref[...] = w1_ref[...] * r1_ref[...] + w2_ref[...] * r2_ref[...]


def _combine_tc(r1, r2, w1n, w2n):
    return pl.pallas_call(
        _combine_tc_body,
        grid=(N_TOK // TOK_BLK,),
        in_specs=[
            pl.BlockSpec((TOK_BLK, D_MODEL), lambda i: (i, 0)),
            pl.BlockSpec((TOK_BLK, D_MODEL), lambda i: (i, 0)),
            pl.BlockSpec((TOK_BLK, 1), lambda i: (i, 0)),
            pl.BlockSpec((TOK_BLK, 1), lambda i: (i, 0)),
        ],
        out_specs=pl.BlockSpec((TOK_BLK, D_MODEL), lambda i: (i, 0)),
        out_shape=jax.ShapeDtypeStruct((N_TOK, D_MODEL), jnp.float32),
    )(r1, r2, w1n, w2n)


def kernel(x, w_gate, w_up, w_down):
    e1, e2, w1n, w2n = _router(x, w_gate)
    pos, st, sg, slo, shi, xs = _dispatch(
        e1.reshape(N_TOK), e2.reshape(N_TOK), x)
    y_sorted = _gmm(xs, w_up, w_down, st, sg, slo, shi)
    r1, r2 = _combine_sc(y_sorted, pos)
    return _combine_tc(r1, r2, w1n, w2n)
